# baseline (speedup 1.0000x reference)
import functools

import jax
import jax.numpy as jnp
from jax import lax
from jax.experimental import pallas as pl
from jax.experimental.pallas import tpu as pltpu
from jax.experimental.pallas import tpu_sc as plsc

_DIM = 32
_NW = 32
_PACK = 128 // _DIM
_HISTP = 256
_GROUP = 8
_L = 16


def _make_gather(batch, hist):
    rows_per_w = batch // _NW
    n_groups = rows_per_w // _GROUP
    orpb = hist * _DIM // 128
    histp8 = (hist + _L - 1) // _L * _L
    n_blocks = histp8 // _L
    mesh = plsc.VectorSubcoreMesh(core_axis_name="c", subcore_axis_name="s")

    @functools.partial(
        pl.kernel,
        out_type=jax.ShapeDtypeStruct((batch * orpb, 128), jnp.float32),
        mesh=mesh,
        scratch_types=[
            pltpu.VMEM((_GROUP, _HISTP), jnp.int32),
            pltpu.VMEM((2, 128), jnp.int32),
            pltpu.VMEM((histp8, 128), jnp.float32),
            pltpu.VMEM((_GROUP * orpb + 8, 128), jnp.float32),
            pltpu.SemaphoreType.DMA,
            pltpu.SemaphoreType.DMA,
            pltpu.SemaphoreType.DMA,
        ],
        compiler_params=pltpu.CompilerParams(
            use_tc_tiling_on_sc=True, needs_layout_passes=False),
    )
    def gather_kernel(idx_hbm, table_hbm, out_hbm, idx_v, gi_v, stage_v,
                      outb_v, idx_sem, gat_sem, wb_sem):
        wid = lax.axis_index("s") * 2 + lax.axis_index("c")
        brow0 = wid * rows_per_w
        iota = lax.iota(jnp.int32, _L)
        drow_off = lax.shift_right_logical(iota, 2)
        dcol_off = jnp.bitwise_and(iota, 3) * _DIM

        def group_body(j, carry):
            base = brow0 + j * _GROUP
            pltpu.async_copy(
                idx_hbm.at[pl.ds(base, _GROUP)], idx_v, idx_sem).wait()

            def chunk_body(c, carry2):
                for v in range(n_blocks):
                    o = v * _L
                    ids = idx_v[c, pl.ds(o, _L)]
                    gi_v[o // 128, pl.ds(o % 128, _L)] = (
                        lax.shift_right_logical(ids, 2))

                g1 = pltpu.async_copy(
                    table_hbm.at[gi_v.at[0]],
                    stage_v.at[pl.ds(0, 128)], gat_sem)
                g2 = pltpu.async_copy(
                    table_hbm.at[gi_v.at[1, pl.ds(0, histp8 - 128)]],
                    stage_v.at[pl.ds(128, histp8 - 128)], gat_sem)
                g1.wait()
                g2.wait()

                def blk_body(r, carry3):
                    t0 = r * _L
                    qv = jnp.bitwise_and(idx_v[c, pl.ds(t0, _L)], 3)
                    colbase = qv * _DIM
                    trow = iota + t0
                    drow = drow_off + (c * orpb + r * _PACK)
                    for jj in range(_DIM):
                        vals = plsc.load_gather(
                            stage_v, [trow, colbase + jj])
                        plsc.store_scatter(
                            outb_v, [drow, dcol_off + jj], vals)
                    return carry3
                lax.fori_loop(0, n_blocks, blk_body, 0)
                return carry2

            lax.fori_loop(0, _GROUP, chunk_body, 0)

            pltpu.async_copy(
                outb_v.at[pl.ds(0, _GROUP * orpb)],
                out_hbm.at[pl.ds(base * orpb, _GROUP * orpb)],
                wb_sem).wait()
            return carry

        lax.fori_loop(0, n_groups, group_body, 0)

    return gather_kernel


def kernel(indices, table):
    b, h = indices.shape
    idx_p = jnp.pad(indices, ((0, 0), (0, _HISTP - h)))
    t4 = table.reshape(table.shape[0] // _PACK, 128)
    out = _make_gather(b, h)(idx_p, t4)
    return out.reshape(b, h, _DIM)


# speedup vs baseline: 2.5054x; 2.5054x over previous
import functools

import jax
import jax.numpy as jnp
from jax import lax
from jax.experimental import pallas as pl
from jax.experimental.pallas import tpu as pltpu
from jax.experimental.pallas import tpu_sc as plsc

_DIM = 32
_NW = 32
_SUB = 128
_N_SUB = 10
_CHUNK = _SUB * _N_SUB
_NBUF = 2


def _make_gather(n_rows):
    b_per_w = n_rows // _NW
    n_chunks = b_per_w // _CHUNK
    assert n_chunks % _NBUF == 0
    mesh = plsc.VectorSubcoreMesh(core_axis_name="c", subcore_axis_name="s")

    @functools.partial(
        pl.kernel,
        out_type=jax.ShapeDtypeStruct((n_rows, _DIM), jnp.float32),
        mesh=mesh,
        scratch_types=[
            pltpu.VMEM((_NBUF, _N_SUB, _SUB), jnp.int32),
            pltpu.VMEM((_NBUF, _CHUNK, _DIM), jnp.float32),
            pltpu.SemaphoreType.DMA((_NBUF,)),
            pltpu.SemaphoreType.DMA((_NBUF,)),
            pltpu.SemaphoreType.DMA((_NBUF,)),
        ],
        compiler_params=pltpu.CompilerParams(use_tc_tiling_on_sc=False),
    )
    def gather_kernel(idx_hbm, table_hbm, out_hbm, idx_v, rows_v,
                      idx_sem, gat_sem, wb_sem):
        wid = lax.axis_index("s") * 2 + lax.axis_index("c")
        idx_row0 = wid * (b_per_w // _SUB)
        out_row0 = wid * b_per_w

        def idx_copy(g, b):
            return pltpu.make_async_copy(
                idx_hbm.at[pl.ds(idx_row0 + g * _N_SUB, _N_SUB)],
                idx_v.at[b], idx_sem.at[b])

        def wb_copy(g, b):
            return pltpu.make_async_copy(
                rows_v.at[b],
                out_hbm.at[pl.ds(out_row0 + g * _CHUNK, _CHUNK)],
                wb_sem.at[b])

        idx_copy(0, 0).start()

        def body(gg, carry):
            for b in range(_NBUF):
                g = gg * _NBUF + b
                idx_copy(g, b).wait()

                @pl.when(g + 1 < n_chunks)
                def _():
                    idx_copy(g + 1, (b + 1) % _NBUF).start()

                @pl.when(g >= _NBUF)
                def _():
                    wb_copy(g - _NBUF, b).wait()

                copies = [
                    pltpu.async_copy(
                        table_hbm.at[idx_v.at[b, j]],
                        rows_v.at[b].at[pl.ds(j * _SUB, _SUB)],
                        gat_sem.at[b],
                    )
                    for j in range(_N_SUB)
                ]
                for c in copies:
                    c.wait()
                wb_copy(g, b).start()
            return carry

        lax.fori_loop(0, n_chunks // _NBUF, body, 0)
        for b in range(_NBUF):
            wb_copy(n_chunks - _NBUF + b, b).wait()

    return gather_kernel


def kernel(indices, table):
    b, h = indices.shape
    n = b * h
    idx2d = indices.reshape(n // _SUB, _SUB)
    out = _make_gather(n)(idx2d, table)
    return out.reshape(b, h, _DIM)
